# baseline (speedup 1.0000x reference)
import jax
import jax.numpy as jnp
from jax.experimental import pallas as pl

TMAX = 20.0
SHARP = 20.0
NOISE = 0.3


def _apply_lin(p, x):
    return x @ p["W"].T + p["b"]


def _apply_mlp(p, x):
    return _apply_lin(p[1], jax.nn.leaky_relu(_apply_lin(p[0], x), 0.1))


def _seg_sum(x, idx, n):
    return jax.ops.segment_sum(x, idx, num_segments=n)


def _seg_mean(x, idx, n):
    s = _seg_sum(x, idx, n)
    c = _seg_sum(jnp.ones((x.shape[0], 1), x.dtype), idx, n)
    return s / jnp.maximum(c, 1.0)


def _bn(p, x):
    m = jnp.mean(x, axis=0)
    v = jnp.mean((x - m) ** 2, axis=0)
    return (x - m) / jnp.sqrt(v + 1e-5) * p["gamma"] + p["beta"]


def _block(p, x_h, x_g, src, tgt, edge_attr, u, batch_e, batch_h, batch_g):
    Nh = x_h.shape[0]; Ng = x_g.shape[0]; Bn = u.shape[0]
    edge_attr = _apply_mlp(p["edge"], jnp.concatenate([x_h[src], x_g[tgt], edge_attr, u[batch_e]], axis=1))
    out = _apply_mlp(p["h1"], jnp.concatenate([x_g[tgt], edge_attr], axis=1))
    nn = _seg_sum(jnp.ones((out.shape[0], 1), out.dtype), src, Nh)
    a = _seg_mean(out, src, Nh)
    b = jnp.sqrt(1e-6 + jax.nn.relu(_seg_mean(out ** 2, src, Nh) - a ** 2))
    c = _seg_mean((out - a[src]) ** 3, src, Nh) / b ** 3
    d = _seg_mean((out - a[src]) ** 4, src, Nh) / b ** 4
    x_h_new = _apply_mlp(p["h2"], jnp.concatenate([x_h, nn, a, b, c, d, u[batch_h]], axis=1))
    outg = _apply_mlp(p["g1"], jnp.concatenate([x_h_new[src], edge_attr], axis=1))
    ag = _seg_sum(outg, tgt, Ng)
    x_g_new = _apply_mlp(p["g2"], jnp.concatenate([x_g, ag, u[batch_g]], axis=1))
    u_new = _apply_mlp(p["glob"], jnp.concatenate([u, _seg_mean(x_h_new, batch_h, Bn), _seg_mean(x_g_new, batch_g, Bn)], axis=1))
    return x_h_new, x_g_new, edge_attr, u_new


def _time_kernel(e_out_ref, noise_ref, time_ref):
    e_out = e_out_ref[...]
    t = jnp.sum(TMAX * jax.nn.sigmoid(e_out), axis=-1)
    t = t + noise_ref[...][:, 0]
    it = jnp.floor(t)
    time_ref[...] = (it + jax.nn.sigmoid(SHARP * (t - 0.5 - it)))[:, None]


def kernel(x_h, x_g, edge_index, edge_attr, u, batch_e, batch_h, batch_g, params):
    src = edge_index[0]; tgt = edge_index[1]
    for p in params["blocks"]:
        x_h, x_g, edge_attr, u = _block(p, x_h, x_g, src, tgt, edge_attr, u, batch_e, batch_h, batch_g)
        x_h = _bn(p["bn_xh"], x_h)
        x_g = _bn(p["bn_xg"], x_g)
        edge_attr = _bn(p["bn_e"], edge_attr)
    e_out = _apply_mlp(params["last"], jnp.concatenate([x_h[src], x_g[tgt], edge_attr, u[batch_e]], axis=1))
    E = e_out.shape[0]
    noise = NOISE * (jax.random.uniform(jax.random.key(42), (E,), dtype=e_out.dtype) - 0.5)
    TE = 5000
    time = pl.pallas_call(
        _time_kernel,
        grid=(E // TE,),
        in_specs=[
            pl.BlockSpec((TE, 1), lambda i: (i, 0)),
            pl.BlockSpec((TE, 1), lambda i: (i, 0)),
        ],
        out_specs=pl.BlockSpec((TE, 1), lambda i: (i, 0)),
        out_shape=jax.ShapeDtypeStruct((E, 1), e_out.dtype),
    )(e_out, noise[:, None])
    return (time[:, 0], edge_index)


# speedup vs baseline: 3.7384x; 3.7384x over previous
import functools
import jax
import jax.numpy as jnp
from jax import lax
from jax.experimental import pallas as pl
from jax.experimental.pallas import tpu as pltpu, tpu_sc as plsc

NC, NS = 2, 16
NW = NC * NS
NP = 50048
NPS = 51200
E = 800000
CH = 200
TB = E // 16
TBB = 2000
TN = 3128
TMAX = 20.0
SHARP = 20.0
NOISE = 0.3

_CP = pltpu.CompilerParams(use_tc_tiling_on_sc=False)
_MESH = plsc.VectorSubcoreMesh(core_axis_name="c", subcore_axis_name="s")
_HIGH = jax.lax.Precision.HIGHEST


def _leaky(x):
    return jnp.where(x > 0, x, 0.1 * x)


@functools.partial(
    pl.kernel, mesh=_MESH, compiler_params=_CP,
    out_type=(jax.ShapeDtypeStruct((E, 16), jnp.float32),
              jax.ShapeDtypeStruct((E, 16), jnp.float32),
              jax.ShapeDtypeStruct((E, 16), jnp.float32)),
    scratch_types=[pltpu.VMEM((CH,), jnp.int32),
                   pltpu.VMEM((CH,), jnp.int32),
                   pltpu.VMEM((CH,), jnp.int32),
                   pltpu.VMEM((CH, 16), jnp.float32),
                   pltpu.VMEM((CH, 16), jnp.float32),
                   pltpu.VMEM((CH, 16), jnp.float32),
                   pltpu.VMEM_SHARED((NP, 16), jnp.float32),
                   pltpu.VMEM_SHARED((NP, 16), jnp.float32),
                   pltpu.VMEM_SHARED((16, 16), jnp.float32),
                   pltpu.SemaphoreType.DMA],
)
def _sc_gather3(xh_hbm, xg_hbm, u_hbm, src_hbm, tgt_hbm, be_hbm,
                gh_hbm, gg_hbm, ue_hbm,
                i1, i2, i3, r1, r2, r3, xh_s, xg_s, u_s, sem):
    c = lax.axis_index("c")
    s = lax.axis_index("s")
    vpt = NP // NS
    pltpu.sync_copy(xh_hbm.at[pl.ds(s * vpt, vpt)], xh_s.at[pl.ds(s * vpt, vpt)])
    pltpu.sync_copy(xg_hbm.at[pl.ds(s * vpt, vpt)], xg_s.at[pl.ds(s * vpt, vpt)])

    @pl.when(s == 0)
    def _():
        pltpu.sync_copy(u_hbm, u_s)
    plsc.subcore_barrier()
    per_tile = E // NW
    base = c * (E // NC) + s * per_tile

    def body(i, carry):
        off = base + i * CH
        pltpu.sync_copy(src_hbm.at[pl.ds(off, CH)], i1)
        pltpu.sync_copy(tgt_hbm.at[pl.ds(off, CH)], i2)
        pltpu.sync_copy(be_hbm.at[pl.ds(off, CH)], i3)
        pltpu.async_copy(xh_s.at[i1], r1, sem).wait()
        pltpu.sync_copy(r1, gh_hbm.at[pl.ds(off, CH)])
        pltpu.async_copy(xg_s.at[i2], r2, sem).wait()
        pltpu.sync_copy(r2, gg_hbm.at[pl.ds(off, CH)])
        pltpu.async_copy(u_s.at[i3], r3, sem).wait()
        pltpu.sync_copy(r3, ue_hbm.at[pl.ds(off, CH)])
        return carry
    lax.fori_loop(0, per_tile // CH, body, 0)


def _make_sc_gather1(D):
    @functools.partial(
        pl.kernel, mesh=_MESH, compiler_params=_CP,
        out_type=jax.ShapeDtypeStruct((E, D), jnp.float32),
        scratch_types=[pltpu.VMEM((CH,), jnp.int32),
                       pltpu.VMEM((CH, D), jnp.float32),
                       pltpu.VMEM_SHARED((NP, D), jnp.float32),
                       pltpu.SemaphoreType.DMA],
    )
    def k(tab_hbm, idx_hbm, out_hbm, iv, rv, tab_s, sem):
        c = lax.axis_index("c")
        s = lax.axis_index("s")
        vpt = NP // NS
        pltpu.sync_copy(tab_hbm.at[pl.ds(s * vpt, vpt)],
                        tab_s.at[pl.ds(s * vpt, vpt)])
        plsc.subcore_barrier()
        per_tile = E // NW
        base = c * (E // NC) + s * per_tile

        def body(i, carry):
            off = base + i * CH
            pltpu.sync_copy(idx_hbm.at[pl.ds(off, CH)], iv)
            pltpu.async_copy(tab_s.at[iv], rv, sem).wait()
            pltpu.sync_copy(rv, out_hbm.at[pl.ds(off, CH)])
            return carry
        lax.fori_loop(0, per_tile // CH, body, 0)
    return k


_sc_gather16 = _make_sc_gather1(16)
_sc_gather32 = _make_sc_gather1(32)


@functools.partial(
    pl.kernel, mesh=_MESH, compiler_params=_CP,
    out_type=(jax.ShapeDtypeStruct((NPS, 16), jnp.float32),
              jax.ShapeDtypeStruct((NPS, 16), jnp.float32)),
    scratch_types=[pltpu.VMEM((CH,), jnp.int32),
                   pltpu.VMEM((CH,), jnp.int32),
                   pltpu.VMEM((CH, 16), jnp.float32),
                   pltpu.VMEM((CH, 16), jnp.float32),
                   pltpu.VMEM_SHARED((16, 16), jnp.float32),
                   pltpu.SemaphoreType.DMA],
)
def _sc_gatheru2(u_hbm, bh_hbm, bg_hbm, uh_hbm, ug_hbm, i1, i2, r1, r2, u_s, sem):
    c = lax.axis_index("c")
    s = lax.axis_index("s")

    @pl.when(s == 0)
    def _():
        pltpu.sync_copy(u_hbm, u_s)
    plsc.subcore_barrier()
    per_tile = NPS // NW
    base = c * (NPS // NC) + s * per_tile

    def body(i, carry):
        off = base + i * CH
        pltpu.sync_copy(bh_hbm.at[pl.ds(off, CH)], i1)
        pltpu.sync_copy(bg_hbm.at[pl.ds(off, CH)], i2)
        pltpu.async_copy(u_s.at[i1], r1, sem).wait()
        pltpu.sync_copy(r1, uh_hbm.at[pl.ds(off, CH)])
        pltpu.async_copy(u_s.at[i2], r2, sem).wait()
        pltpu.sync_copy(r2, ug_hbm.at[pl.ds(off, CH)])
        return carry
    lax.fori_loop(0, per_tile // CH, body, 0)


@functools.partial(
    pl.kernel, mesh=_MESH, compiler_params=_CP,
    out_type=jax.ShapeDtypeStruct((NC, NP, 32), jnp.float32),
    scratch_types=[pltpu.VMEM((CH,), jnp.int32),
                   pltpu.VMEM((CH, 32), jnp.float32),
                   pltpu.VMEM_SHARED((NP, 32), jnp.float32),
                   pltpu.SemaphoreType.DMA],
)
def _sc_scatter_pair(va_hbm, vb_hbm, idx_hbm, z_hbm, out_hbm, iv, vv, acc, sem):
    c = lax.axis_index("c")
    s = lax.axis_index("s")
    rpt = NP // NS
    pltpu.sync_copy(z_hbm, acc.at[pl.ds(s * rpt, rpt)])
    plsc.subcore_barrier()
    per_tile = E // NS
    base = s * per_tile

    def body_a(i, carry):
        off = base + i * CH
        pltpu.sync_copy(idx_hbm.at[pl.ds(off, CH)], iv)
        pltpu.sync_copy(va_hbm.at[pl.ds(off, CH)], vv)
        pltpu.sync_copy(vv, acc.at[iv], add=True)
        return carry

    def body_b(i, carry):
        off = base + i * CH
        pltpu.sync_copy(idx_hbm.at[pl.ds(off, CH)], iv)
        pltpu.sync_copy(vb_hbm.at[pl.ds(off, CH)], vv)
        pltpu.sync_copy(vv, acc.at[iv], add=True)
        return carry

    @pl.when(c == 0)
    def _():
        lax.fori_loop(0, per_tile // CH, body_a, 0)

    @pl.when(c == 1)
    def _():
        lax.fori_loop(0, per_tile // CH, body_b, 0)
    plsc.subcore_barrier()
    pltpu.sync_copy(acc.at[pl.ds(s * rpt, rpt)], out_hbm.at[c, pl.ds(s * rpt, rpt)])


def _make_sc_scatter1(N, D, L, ones_mode):
    rpt = N // NS
    per_core = L // NC
    per_tile = per_core // NS
    assert per_tile % CH == 0 and rpt % 8 == 0

    @functools.partial(
        pl.kernel, mesh=_MESH, compiler_params=_CP,
        out_type=jax.ShapeDtypeStruct((NC, N, D), jnp.float32),
        scratch_types=[pltpu.VMEM((CH,), jnp.int32),
                       pltpu.VMEM((CH, D), jnp.float32),
                       pltpu.VMEM_SHARED((N, D), jnp.float32),
                       pltpu.SemaphoreType.DMA],
    )
    def k(vals_hbm, idx_hbm, z_hbm, out_hbm, iv, vv, acc, sem):
        c = lax.axis_index("c")
        s = lax.axis_index("s")
        pltpu.sync_copy(z_hbm, acc.at[pl.ds(s * rpt, rpt)])
        if ones_mode:
            pltpu.sync_copy(vals_hbm, vv)
        plsc.subcore_barrier()
        base = c * per_core + s * per_tile

        def body(i, carry):
            off = base + i * CH
            pltpu.sync_copy(idx_hbm.at[pl.ds(off, CH)], iv)
            if not ones_mode:
                pltpu.sync_copy(vals_hbm.at[pl.ds(off, CH)], vv)
            pltpu.sync_copy(vv, acc.at[iv], add=True)
            return carry
        lax.fori_loop(0, per_tile // CH, body, 0)
        plsc.subcore_barrier()
        pltpu.sync_copy(acc.at[pl.ds(s * rpt, rpt)], out_hbm.at[c, pl.ds(s * rpt, rpt)])
    return k


_sc_scat_edges32 = _make_sc_scatter1(NP, 32, E, False)
_sc_scat_ones_src = _make_sc_scatter1(NP, 16, E, True)
_sc_scat_node16 = _make_sc_scatter1(128, 16, NPS, False)
_sc_scat_ones_b = _make_sc_scatter1(128, 16, NPS, True)


def _dot(a, b):
    return jax.lax.dot(a, b, precision=_HIGH)


def _ek1_body(ghp, ggp, eap, uep, kh, kg, ke, ku, b1, w2, b2,
              w3g, w3e, b3, w4, b4, e2o, outo, sqo):
    z1 = (_dot(ghp[...], kh[...]) + _dot(ggp[...], kg[...]) +
          _dot(eap[...], ke[...]) + _dot(uep[...], ku[...]) + b1[...])
    e2 = _dot(_leaky(z1), w2[...]) + b2[...]
    z3 = _dot(ggp[...], w3g[...]) + _dot(e2, w3e[...]) + b3[...]
    out = _dot(_leaky(z3), w4[...]) + b4[...]
    e2o[...] = e2
    outo[...] = out
    sqo[...] = out * out


def _tc_ek1(ghp, ggp, eap, uep, kh, kg, ke, ku, b1, w2, b2, w3g, w3e, b3, w4, b4):
    n = TB // TBB
    row = lambda i: (i, 0)
    fix = lambda i: (0, 0)
    bs = lambda shp, im: pl.BlockSpec(shp, im)
    return pl.pallas_call(
        _ek1_body,
        grid=(n,),
        in_specs=[bs((TBB, 256), row)] * 4 + [
            bs((256, 256), fix), bs((256, 256), fix), bs((256, 256), fix),
            bs((256, 256), fix), bs((1, 256), fix), bs((256, 256), fix),
            bs((1, 256), fix), bs((256, 512), fix), bs((256, 512), fix),
            bs((1, 512), fix), bs((512, 512), fix), bs((1, 512), fix)],
        out_specs=[bs((TBB, 256), row), bs((TBB, 512), row), bs((TBB, 512), row)],
        out_shape=[jax.ShapeDtypeStruct((TB, 256), jnp.float32),
                   jax.ShapeDtypeStruct((TB, 512), jnp.float32),
                   jax.ShapeDtypeStruct((TB, 512), jnp.float32)],
    )(ghp, ggp, eap, uep, kh, kg, ke, ku, b1, w2, b2, w3g, w3e, b3, w4, b4)


def _ekc_body(outp, agp, t3o, t4o):
    t = outp[...] - agp[...]
    t2 = t * t
    t3o[...] = t2 * t
    t4o[...] = t2 * t2


def _tc_ekc(outp, agp):
    n = TB // TBB
    bs = pl.BlockSpec((TBB, 512), lambda i: (i, 0))
    return pl.pallas_call(
        _ekc_body, grid=(n,), in_specs=[bs, bs], out_specs=[bs, bs],
        out_shape=[jax.ShapeDtypeStruct((TB, 512), jnp.float32)] * 2,
    )(outp, agp)


def _ek2_body(gh2p, e2p, k5h, k5e, b5, w6, b6, outgo, stato, accs):
    i = pl.program_id(0)
    z = _dot(gh2p[...], k5h[...]) + _dot(e2p[...], k5e[...]) + b5[...]
    outgo[...] = _dot(_leaky(z), w6[...]) + b6[...]
    e2 = e2p[...]

    @pl.when(i == 0)
    def _():
        accs[...] = jnp.zeros_like(accs)
    accs[0:1, :] += jnp.sum(e2, axis=0, keepdims=True)
    accs[1:2, :] += jnp.sum(e2 * e2, axis=0, keepdims=True)

    @pl.when(i == pl.num_programs(0) - 1)
    def _():
        stato[...] = accs[...]


def _tc_ek2(gh2p, e2p, k5h, k5e, b5, w6, b6):
    n = TB // TBB
    row = lambda i: (i, 0)
    fix = lambda i: (0, 0)
    bs = lambda shp, im: pl.BlockSpec(shp, im)
    return pl.pallas_call(
        _ek2_body, grid=(n,),
        in_specs=[bs((TBB, 256), row), bs((TBB, 256), row),
                  bs((256, 512), fix), bs((256, 512), fix), bs((1, 512), fix),
                  bs((512, 512), fix), bs((1, 512), fix)],
        out_specs=[bs((TBB, 512), row), bs((8, 256), fix)],
        out_shape=[jax.ShapeDtypeStruct((TB, 512), jnp.float32),
                   jax.ShapeDtypeStruct((8, 256), jnp.float32)],
        scratch_shapes=[pltpu.VMEM((8, 256), jnp.float32)],
    )(gh2p, e2p, k5h, k5e, b5, w6, b6)


def _nodea_body(sv, c0, c1, ao):
    cnt = jnp.maximum((c0[...] + c1[...])[:, :1], 1.0)
    ao[...] = sv[...] / cnt


def _tc_nodea(sv, c0, c1):
    n = NP // TN
    row = lambda i: (i, 0)
    return pl.pallas_call(
        _nodea_body, grid=(n,),
        in_specs=[pl.BlockSpec((TN, 32), row), pl.BlockSpec((TN, 16), row),
                  pl.BlockSpec((TN, 16), row)],
        out_specs=pl.BlockSpec((TN, 32), row),
        out_shape=jax.ShapeDtypeStruct((NP, 32), jnp.float32),
    )(sv, c0, c1)


def _nodeb_body(xh, c0, c1, ss, t3s, t4s, a, uh,
                ah, aa, ab, ac, ad, au, wn, b1, w2, b2, xo, sto, accs):
    i = pl.program_id(0)
    cnt = (c0[...] + c1[...])[:, :1]
    cm = jnp.maximum(cnt, 1.0)
    av = a[...]
    bv = jnp.sqrt(1e-6 + jnp.maximum(ss[...] / cm - av * av, 0.0))
    b2v = bv * bv
    cv = (t3s[...] / cm) / (b2v * bv)
    dv = (t4s[...] / cm) / (b2v * b2v)
    z = (_dot(xh[...], ah[...]) + _dot(av, aa[...]) + _dot(bv, ab[...]) +
         _dot(cv, ac[...]) + _dot(dv, ad[...]) + _dot(uh[...], au[...]) +
         cnt * wn[...] + b1[...])
    xn = _dot(_leaky(z), w2[...]) + b2[...]
    xo[...] = xn
    rows = i * TN + jax.lax.broadcasted_iota(jnp.int32, (TN, 1), 0)
    m = (rows < 50000).astype(jnp.float32)
    xm = xn * m

    @pl.when(i == 0)
    def _():
        accs[...] = jnp.zeros_like(accs)
    accs[0:1, :] += jnp.sum(xm, axis=0, keepdims=True)
    accs[1:2, :] += jnp.sum(xm * xn, axis=0, keepdims=True)

    @pl.when(i == pl.num_programs(0) - 1)
    def _():
        sto[...] = accs[...]


def _tc_nodeb(xh, c0, c1, ss, t3s, t4s, a, uh, ah, aa, ab, ac, ad, au, wn, b1, w2, b2):
    n = NP // TN
    row = lambda i: (i, 0)
    fix = lambda i: (0, 0)
    bs = lambda shp, im: pl.BlockSpec(shp, im)
    return pl.pallas_call(
        _nodeb_body, grid=(n,),
        in_specs=[bs((TN, 16), row), bs((TN, 16), row), bs((TN, 16), row),
                  bs((TN, 32), row), bs((TN, 32), row), bs((TN, 32), row),
                  bs((TN, 32), row), bs((TN, 16), row),
                  bs((16, 16), fix), bs((32, 16), fix), bs((32, 16), fix),
                  bs((32, 16), fix), bs((32, 16), fix), bs((16, 16), fix),
                  bs((1, 16), fix), bs((1, 16), fix), bs((16, 16), fix),
                  bs((1, 16), fix)],
        out_specs=[bs((TN, 16), row), bs((8, 16), fix)],
        out_shape=[jax.ShapeDtypeStruct((NP, 16), jnp.float32),
                   jax.ShapeDtypeStruct((8, 16), jnp.float32)],
        scratch_shapes=[pltpu.VMEM((8, 16), jnp.float32)],
    )(xh, c0, c1, ss, t3s, t4s, a, uh, ah, aa, ab, ac, ad, au, wn, b1, w2, b2)


def _nodec_body(xg, ag0, ag1, ug, axg, aag, aug, b1, w2, b2, xo, sto, accs):
    i = pl.program_id(0)
    agv = ag0[...] + ag1[...]
    z = (_dot(xg[...], axg[...]) + _dot(agv, aag[...]) +
         _dot(ug[...], aug[...]) + b1[...])
    xn = _dot(_leaky(z), w2[...]) + b2[...]
    xo[...] = xn
    rows = i * TN + jax.lax.broadcasted_iota(jnp.int32, (TN, 1), 0)
    m = (rows < 50000).astype(jnp.float32)
    xm = xn * m

    @pl.when(i == 0)
    def _():
        accs[...] = jnp.zeros_like(accs)
    accs[0:1, :] += jnp.sum(xm, axis=0, keepdims=True)
    accs[1:2, :] += jnp.sum(xm * xn, axis=0, keepdims=True)

    @pl.when(i == pl.num_programs(0) - 1)
    def _():
        sto[...] = accs[...]


def _tc_nodec(xg, ag0, ag1, ug, axg, aag, aug, b1, w2, b2):
    n = NP // TN
    row = lambda i: (i, 0)
    fix = lambda i: (0, 0)
    bs = lambda shp, im: pl.BlockSpec(shp, im)
    return pl.pallas_call(
        _nodec_body, grid=(n,),
        in_specs=[bs((TN, 16), row), bs((TN, 32), row), bs((TN, 32), row),
                  bs((TN, 16), row),
                  bs((16, 16), fix), bs((32, 16), fix), bs((16, 16), fix),
                  bs((1, 16), fix), bs((16, 16), fix), bs((1, 16), fix)],
        out_specs=[bs((TN, 16), row), bs((8, 16), fix)],
        out_shape=[jax.ShapeDtypeStruct((NP, 16), jnp.float32),
                   jax.ShapeDtypeStruct((8, 16), jnp.float32)],
        scratch_shapes=[pltpu.VMEM((8, 16), jnp.float32)],
    )(xg, ag0, ag1, ug, axg, aag, aug, b1, w2, b2)


def _glob_body(u, hs0, hs1, cb0, cb1, gs0, gs1, cg0, cg1,
               au, ahm, agm, b1, w2, b2, uo):
    hm = (hs0[...] + hs1[...])[:16] / jnp.maximum((cb0[...] + cb1[...])[:16, :1], 1.0)
    gm = (gs0[...] + gs1[...])[:16] / jnp.maximum((cg0[...] + cg1[...])[:16, :1], 1.0)
    z = _dot(u[...], au[...]) + _dot(hm, ahm[...]) + _dot(gm, agm[...]) + b1[...]
    uo[...] = _dot(_leaky(z), w2[...]) + b2[...]


def _tc_glob(u, hs0, hs1, cb0, cb1, gs0, gs1, cg0, cg1, au, ahm, agm, b1, w2, b2):
    full = lambda shp: pl.BlockSpec(shp, lambda: (0, 0))
    return pl.pallas_call(
        _glob_body,
        in_specs=[full((16, 16))] + [full((128, 16))] * 8 +
                 [full((16, 16)), full((16, 16)), full((16, 16)),
                  full((1, 16)), full((16, 16)), full((1, 16))],
        out_specs=full((16, 16)),
        out_shape=jax.ShapeDtypeStruct((16, 16), jnp.float32),
    )(u, hs0, hs1, cb0, cb1, gs0, gs1, cg0, cg1, au, ahm, agm, b1, w2, b2)


def _ekf_body(ghp, ggp, eap, uep, nz, kh, kg, ke, ku, b1, w2, b2, to):
    z1 = (_dot(ghp[...], kh[...]) + _dot(ggp[...], kg[...]) +
          _dot(eap[...], ke[...]) + _dot(uep[...], ku[...]) + b1[...])
    e_out = _leaky(z1) * w2[...] + b2[...]
    t = TMAX * jax.nn.sigmoid(e_out) + nz[...]
    it = jnp.floor(t)
    to[...] = it + jax.nn.sigmoid(SHARP * (t - 0.5 - it))


def _tc_ekf(ghp, ggp, eap, uep, nz, kh, kg, ke, ku, b1, w2, b2):
    n = TB // TBB
    row = lambda i: (i, 0)
    fix = lambda i: (0, 0)
    bs = lambda shp, im: pl.BlockSpec(shp, im)
    return pl.pallas_call(
        _ekf_body, grid=(n,),
        in_specs=[bs((TBB, 256), row)] * 4 + [bs((TBB, 16), row)] +
                 [bs((256, 16), fix)] * 4 +
                 [bs((1, 16), fix), bs((1, 16), fix), bs((1, 16), fix)],
        out_specs=bs((TBB, 16), row),
        out_shape=jax.ShapeDtypeStruct((TB, 16), jnp.float32),
    )(ghp, ggp, eap, uep, nz, kh, kg, ke, ku, b1, w2, b2)


_I16 = None


def _kron16(w):
    din, dout = w.shape
    z = jnp.zeros((16, din, 16, dout), w.dtype)
    idx = jnp.arange(16)
    z = z.at[idx, :, idx, :].set(w)
    return z.reshape(16 * din, 16 * dout)


def _pk(x):
    return x.reshape(TB, -1)


def _unpk(x, d):
    return x.reshape(E, d)


def _stats_to_affine(s1, s2, n, gamma, beta):
    m = s1 / n
    v = s2 / n - m * m
    sc = gamma / jnp.sqrt(v + 1e-5)
    return sc, beta - m * sc


def kernel(x_h, x_g, edge_index, edge_attr, u, batch_e, batch_h, batch_g, params):
    f32 = jnp.float32
    src = edge_index[0].astype(jnp.int32)
    tgt = edge_index[1].astype(jnp.int32)
    be = batch_e.astype(jnp.int32)
    pad_n = NP - x_h.shape[0]
    xh = jnp.pad(x_h.astype(f32), ((0, pad_n), (0, 0)))
    xg = jnp.pad(x_g.astype(f32), ((0, pad_n), (0, 0)))
    up = jnp.pad(u.astype(f32), ((0, 16 - u.shape[0]), (0, 0)))
    ea = edge_attr.astype(f32)
    bh_pad = jnp.pad(batch_h.astype(jnp.int32), (0, NPS - batch_h.shape[0]),
                     constant_values=127)
    bg_pad = jnp.pad(batch_g.astype(jnp.int32), (0, NPS - batch_g.shape[0]),
                     constant_values=127)

    z32 = jnp.zeros((NP // NS, 32), f32)
    z16 = jnp.zeros((NP // NS, 16), f32)
    z16s = jnp.zeros((128 // NS, 16), f32)
    ones16 = jnp.ones((CH, 16), f32)

    cnt = _sc_scat_ones_src(ones16, src, z16)
    c0, c1 = cnt[0], cnt[1]
    cbh = _sc_scat_ones_b(ones16, bh_pad, z16s)
    cbg = _sc_scat_ones_b(ones16, bg_pad, z16s)

    one = jnp.ones((16,), f32)
    zero = jnp.zeros((16,), f32)
    sh, th = one, zero
    sg, tg = one, zero
    se, te = one, zero

    eap = _pk(ea)
    n_real = jnp.float32(50000.0)

    for p in params["blocks"]:
        w1 = p["edge"][0]["W"]; b1 = p["edge"][0]["b"]
        w2 = p["edge"][1]["W"]; b2 = p["edge"][1]["b"]
        w3 = p["h1"][0]["W"]; b3 = p["h1"][0]["b"]
        w4 = p["h1"][1]["W"]; b4 = p["h1"][1]["b"]

        gh, gg, ue = _sc_gather3(xh, xg, up, src, tgt, be)
        ghp, ggp, uep = _pk(gh), _pk(gg), _pk(ue)

        w1h = w1[:, 0:16].T * sh[:, None]
        w1g = w1[:, 16:32].T * sg[:, None]
        w1e = w1[:, 32:48].T * se[:, None]
        w1u = w1[:, 48:64].T
        bias1 = b1 + th @ w1[:, 0:16].T + tg @ w1[:, 16:32].T + te @ w1[:, 32:48].T
        kh = _kron16(w1h); kg = _kron16(w1g); ke = _kron16(w1e); ku = _kron16(w1u)
        b1p = jnp.tile(bias1, 16)[None, :]
        w2p = _kron16(w2.T); b2p = jnp.tile(b2, 16)[None, :]
        w3g = w3[:, 0:16].T * sg[:, None]
        w3e = w3[:, 16:32].T
        bias3 = b3 + tg @ w3[:, 0:16].T
        k3g = _kron16(w3g); k3e = _kron16(w3e)
        b3p = jnp.tile(bias3, 16)[None, :]
        w4p = _kron16(w4.T); b4p = jnp.tile(b4, 16)[None, :]

        e2p, outp, sqp = _tc_ek1(ghp, ggp, eap, uep, kh, kg, ke, ku, b1p,
                                 w2p, b2p, k3g, k3e, b3p, w4p, b4p)

        out_e = _unpk(outp, 32)
        sq_e = _unpk(sqp, 32)
        s1pair = _sc_scatter_pair(out_e, sq_e, src, z32)
        a = _tc_nodea(s1pair[0], c0, c1)
        ag = _sc_gather32(a, src)
        t3p, t4p = _tc_ekc(outp, _pk(ag))
        s2pair = _sc_scatter_pair(_unpk(t3p, 32), _unpk(t4p, 32), src, z32)

        uh, ug = _sc_gatheru2(up, bh_pad, bg_pad)

        w5 = p["h2"][0]["W"]; b5 = p["h2"][0]["b"]
        w6 = p["h2"][1]["W"]; b6 = p["h2"][1]["b"]
        ah = w5[:, 0:16].T * sh[:, None]
        wn = w5[:, 16:17].T
        aa = w5[:, 17:49].T
        ab = w5[:, 49:81].T
        ac = w5[:, 81:113].T
        ad = w5[:, 113:145].T
        au = w5[:, 145:161].T
        bias5 = (b5 + th @ w5[:, 0:16].T)[None, :]
        xhn, sth = _tc_nodeb(xh, c0, c1, s1pair[1], s2pair[0], s2pair[1], a,
                             uh[:NP], ah, aa, ab, ac, ad, au, wn, bias5,
                             w6.T, b6[None, :])

        gh2 = _sc_gather16(xhn, src)
        w7 = p["g1"][0]["W"]; b7 = p["g1"][0]["b"]
        w8 = p["g1"][1]["W"]; b8 = p["g1"][1]["b"]
        k7h = _kron16(w7[:, 0:16].T)
        k7e = _kron16(w7[:, 16:32].T)
        b7p = jnp.tile(b7, 16)[None, :]
        w8p = _kron16(w8.T); b8p = jnp.tile(b8, 16)[None, :]
        outgp, ste = _tc_ek2(_pk(gh2), e2p, k7h, k7e, b7p, w8p, b8p)
        agp_pair = _sc_scat_edges32(_unpk(outgp, 32), tgt, z32)

        w9 = p["g2"][0]["W"]; b9 = p["g2"][0]["b"]
        wa = p["g2"][1]["W"]; ba = p["g2"][1]["b"]
        axg = w9[:, 0:16].T * sg[:, None]
        aag = w9[:, 16:48].T
        aug = w9[:, 48:64].T
        bias9 = (b9 + tg @ w9[:, 0:16].T)[None, :]
        xgn, stg = _tc_nodec(xg, agp_pair[0], agp_pair[1], ug[:NP],
                             axg, aag, aug, bias9, wa.T, ba[None, :])

        xhn_s = jnp.pad(xhn, ((0, NPS - NP), (0, 0)))
        xgn_s = jnp.pad(xgn, ((0, NPS - NP), (0, 0)))
        hsum = _sc_scat_node16(xhn_s, bh_pad, z16s)
        gsum = _sc_scat_node16(xgn_s, bg_pad, z16s)
        wg1 = p["glob"][0]["W"]; bg1 = p["glob"][0]["b"]
        wg2 = p["glob"][1]["W"]; bg2 = p["glob"][1]["b"]
        au_g = wg1[:, 0:16].T
        ahm = wg1[:, 16:32].T
        agm = wg1[:, 32:48].T
        u_new = _tc_glob(up, hsum[0], hsum[1], cbh[0], cbh[1],
                         gsum[0], gsum[1], cbg[0], cbg[1],
                         au_g, ahm, agm, bg1[None, :], wg2.T, bg2[None, :])

        es = ste[0].reshape(16, 16).sum(axis=0)
        ess = ste[1].reshape(16, 16).sum(axis=0)
        sh, th = _stats_to_affine(sth[0], sth[1], n_real, p["bn_xh"]["gamma"], p["bn_xh"]["beta"])
        sg, tg = _stats_to_affine(stg[0], stg[1], n_real, p["bn_xg"]["gamma"], p["bn_xg"]["beta"])
        se, te = _stats_to_affine(es, ess, jnp.float32(E), p["bn_e"]["gamma"], p["bn_e"]["beta"])

        xh, xg, up = xhn, xgn, u_new
        eap = e2p

    gh, gg, ue = _sc_gather3(xh, xg, up, src, tgt, be)
    wl1 = params["last"][0]["W"]; bl1 = params["last"][0]["b"]
    wl2 = params["last"][1]["W"]; bl2 = params["last"][1]["b"]
    khf = _kron16(wl1[:, 0:16].T * sh[:, None])
    kgf = _kron16(wl1[:, 16:32].T * sg[:, None])
    kef = _kron16(wl1[:, 32:48].T * se[:, None])
    kuf = _kron16(wl1[:, 48:64].T)
    biasf = (bl1 + th @ wl1[:, 0:16].T + tg @ wl1[:, 16:32].T
             + te @ wl1[:, 32:48].T)
    b1f = jnp.tile(biasf, 16)[None, :]
    w2f = jnp.tile(wl2[0], 16)[None, :]
    b2f = jnp.tile(bl2, 16)[None, :]
    noise = NOISE * (jax.random.uniform(jax.random.key(42), (E,), dtype=f32) - 0.5)
    tpk = _tc_ekf(_pk(gh), _pk(gg), eap, _pk(ue), noise.reshape(TB, 16),
                  khf, kgf, kef, kuf, b1f, w2f, b2f)
    return (tpk.reshape(E), edge_index)


# speedup vs baseline: 4.3291x; 1.1580x over previous
import functools
import jax
import jax.numpy as jnp
from jax import lax
from jax.experimental import pallas as pl
from jax.experimental.pallas import tpu as pltpu, tpu_sc as plsc

NC, NS = 2, 16
NW = NC * NS
NP = 50048
NPS = 51200
E = 800000
CH = 200
TB = E // 16
TBB = 2000
TN = 3128
TMAX = 20.0
SHARP = 20.0
NOISE = 0.3

_CP = pltpu.CompilerParams(use_tc_tiling_on_sc=False)
_MESH = plsc.VectorSubcoreMesh(core_axis_name="c", subcore_axis_name="s")
_HIGH = jax.lax.Precision.HIGHEST


def _leaky(x):
    return jnp.where(x > 0, x, 0.1 * x)


@functools.partial(
    pl.kernel, mesh=_MESH, compiler_params=_CP,
    out_type=(jax.ShapeDtypeStruct((E, 16), jnp.float32),
              jax.ShapeDtypeStruct((E, 16), jnp.float32),
              jax.ShapeDtypeStruct((E, 16), jnp.float32)),
    scratch_types=[pltpu.VMEM((CH,), jnp.int32),
                   pltpu.VMEM((CH,), jnp.int32),
                   pltpu.VMEM((CH,), jnp.int32),
                   pltpu.VMEM((CH, 16), jnp.float32),
                   pltpu.VMEM((CH, 16), jnp.float32),
                   pltpu.VMEM((CH, 16), jnp.float32),
                   pltpu.VMEM_SHARED((NP, 16), jnp.float32),
                   pltpu.VMEM_SHARED((NP, 16), jnp.float32),
                   pltpu.VMEM_SHARED((16, 16), jnp.float32),
                   pltpu.SemaphoreType.DMA],
)
def _sc_gather3(xh_hbm, xg_hbm, u_hbm, src_hbm, tgt_hbm, be_hbm,
                gh_hbm, gg_hbm, ue_hbm,
                i1, i2, i3, r1, r2, r3, xh_s, xg_s, u_s, sem):
    c = lax.axis_index("c")
    s = lax.axis_index("s")
    vpt = NP // NS
    pltpu.sync_copy(xh_hbm.at[pl.ds(s * vpt, vpt)], xh_s.at[pl.ds(s * vpt, vpt)])
    pltpu.sync_copy(xg_hbm.at[pl.ds(s * vpt, vpt)], xg_s.at[pl.ds(s * vpt, vpt)])

    @pl.when(s == 0)
    def _():
        pltpu.sync_copy(u_hbm, u_s)
    plsc.subcore_barrier()
    per_tile = E // NW
    base = c * (E // NC) + s * per_tile

    def body(i, carry):
        off = base + i * CH
        pltpu.sync_copy(src_hbm.at[pl.ds(off, CH)], i1)
        pltpu.sync_copy(tgt_hbm.at[pl.ds(off, CH)], i2)
        pltpu.sync_copy(be_hbm.at[pl.ds(off, CH)], i3)
        pltpu.async_copy(xh_s.at[i1], r1, sem).wait()
        pltpu.sync_copy(r1, gh_hbm.at[pl.ds(off, CH)])
        pltpu.async_copy(xg_s.at[i2], r2, sem).wait()
        pltpu.sync_copy(r2, gg_hbm.at[pl.ds(off, CH)])
        pltpu.async_copy(u_s.at[i3], r3, sem).wait()
        pltpu.sync_copy(r3, ue_hbm.at[pl.ds(off, CH)])
        return carry
    lax.fori_loop(0, per_tile // CH, body, 0)


def _make_sc_gather1(D, ch, V):
    @functools.partial(
        pl.kernel, mesh=_MESH, compiler_params=_CP,
        out_type=jax.ShapeDtypeStruct((E, D), jnp.float32),
        scratch_types=[pltpu.VMEM((ch,), jnp.int32),
                       pltpu.VMEM((ch, D), jnp.float32),
                       pltpu.VMEM_SHARED((V, D), jnp.float32),
                       pltpu.SemaphoreType.DMA],
    )
    def k(tab_hbm, idx_hbm, out_hbm, iv, rv, tab_s, sem):
        c = lax.axis_index("c")
        s = lax.axis_index("s")
        vpt = V // NS
        if vpt >= 8:
            pltpu.sync_copy(tab_hbm.at[pl.ds(s * vpt, vpt)],
                            tab_s.at[pl.ds(s * vpt, vpt)])
        else:
            @pl.when(s == 0)
            def _():
                pltpu.sync_copy(tab_hbm, tab_s)
        plsc.subcore_barrier()
        per_tile = E // NW
        base = c * (E // NC) + s * per_tile

        def body(i, carry):
            off = base + i * ch
            pltpu.sync_copy(idx_hbm.at[pl.ds(off, ch)], iv)
            pltpu.async_copy(tab_s.at[iv], rv, sem).wait()
            pltpu.sync_copy(rv, out_hbm.at[pl.ds(off, ch)])
            return carry
        lax.fori_loop(0, per_tile // ch, body, 0)
    return k


_sc_gather16 = _make_sc_gather1(16, 1000, NP)
_sc_gather32 = _make_sc_gather1(32, 200, NP)
_sc_gatherue = _make_sc_gather1(16, 1000, 16)


@functools.partial(
    pl.kernel, mesh=_MESH, compiler_params=_CP,
    out_type=(jax.ShapeDtypeStruct((NPS, 16), jnp.float32),
              jax.ShapeDtypeStruct((NPS, 16), jnp.float32)),
    scratch_types=[pltpu.VMEM((CH,), jnp.int32),
                   pltpu.VMEM((CH,), jnp.int32),
                   pltpu.VMEM((CH, 16), jnp.float32),
                   pltpu.VMEM((CH, 16), jnp.float32),
                   pltpu.VMEM_SHARED((16, 16), jnp.float32),
                   pltpu.SemaphoreType.DMA],
)
def _sc_gatheru2(u_hbm, bh_hbm, bg_hbm, uh_hbm, ug_hbm, i1, i2, r1, r2, u_s, sem):
    c = lax.axis_index("c")
    s = lax.axis_index("s")

    @pl.when(s == 0)
    def _():
        pltpu.sync_copy(u_hbm, u_s)
    plsc.subcore_barrier()
    per_tile = NPS // NW
    base = c * (NPS // NC) + s * per_tile

    def body(i, carry):
        off = base + i * CH
        pltpu.sync_copy(bh_hbm.at[pl.ds(off, CH)], i1)
        pltpu.sync_copy(bg_hbm.at[pl.ds(off, CH)], i2)
        pltpu.async_copy(u_s.at[i1], r1, sem).wait()
        pltpu.sync_copy(r1, uh_hbm.at[pl.ds(off, CH)])
        pltpu.async_copy(u_s.at[i2], r2, sem).wait()
        pltpu.sync_copy(r2, ug_hbm.at[pl.ds(off, CH)])
        return carry
    lax.fori_loop(0, per_tile // CH, body, 0)


CHP = 400


@functools.partial(
    pl.kernel, mesh=_MESH, compiler_params=_CP,
    out_type=jax.ShapeDtypeStruct((NC, NP, 32), jnp.float32),
    scratch_types=[pltpu.VMEM((CHP,), jnp.int32),
                   pltpu.VMEM((CHP, 32), jnp.float32),
                   pltpu.VMEM_SHARED((NP, 32), jnp.float32),
                   pltpu.SemaphoreType.DMA],
)
def _sc_scatter_pair(va_hbm, vb_hbm, idx_hbm, z_hbm, out_hbm, iv, vv, acc, sem):
    c = lax.axis_index("c")
    s = lax.axis_index("s")
    rpt = NP // NS
    pltpu.sync_copy(z_hbm, acc.at[pl.ds(s * rpt, rpt)])
    plsc.subcore_barrier()
    per_tile = E // NS
    base = s * per_tile

    def body_a(i, carry):
        off = base + i * CHP
        pltpu.sync_copy(idx_hbm.at[pl.ds(off, CHP)], iv)
        pltpu.sync_copy(va_hbm.at[pl.ds(off, CHP)], vv)
        pltpu.sync_copy(vv, acc.at[iv], add=True)
        return carry

    def body_b(i, carry):
        off = base + i * CHP
        pltpu.sync_copy(idx_hbm.at[pl.ds(off, CHP)], iv)
        pltpu.sync_copy(vb_hbm.at[pl.ds(off, CHP)], vv)
        pltpu.sync_copy(vv, acc.at[iv], add=True)
        return carry

    @pl.when(c == 0)
    def _():
        lax.fori_loop(0, per_tile // CHP, body_a, 0)

    @pl.when(c == 1)
    def _():
        lax.fori_loop(0, per_tile // CHP, body_b, 0)
    plsc.subcore_barrier()
    pltpu.sync_copy(acc.at[pl.ds(s * rpt, rpt)], out_hbm.at[c, pl.ds(s * rpt, rpt)])


def _make_sc_scatter1(N, D, L, ones_mode, ch):
    rpt = N // NS
    per_core = L // NC
    per_tile = per_core // NS
    assert per_tile % ch == 0 and rpt % 8 == 0

    @functools.partial(
        pl.kernel, mesh=_MESH, compiler_params=_CP,
        out_type=jax.ShapeDtypeStruct((NC, N, D), jnp.float32),
        scratch_types=[pltpu.VMEM((ch,), jnp.int32),
                       pltpu.VMEM((ch, D), jnp.float32),
                       pltpu.VMEM_SHARED((N, D), jnp.float32),
                       pltpu.SemaphoreType.DMA],
    )
    def k(vals_hbm, idx_hbm, z_hbm, out_hbm, iv, vv, acc, sem):
        c = lax.axis_index("c")
        s = lax.axis_index("s")
        pltpu.sync_copy(z_hbm, acc.at[pl.ds(s * rpt, rpt)])
        if ones_mode:
            pltpu.sync_copy(vals_hbm, vv)
        plsc.subcore_barrier()
        base = c * per_core + s * per_tile

        def body(i, carry):
            off = base + i * ch
            pltpu.sync_copy(idx_hbm.at[pl.ds(off, ch)], iv)
            if not ones_mode:
                pltpu.sync_copy(vals_hbm.at[pl.ds(off, ch)], vv)
            pltpu.sync_copy(vv, acc.at[iv], add=True)
            return carry
        lax.fori_loop(0, per_tile // ch, body, 0)
        plsc.subcore_barrier()
        pltpu.sync_copy(acc.at[pl.ds(s * rpt, rpt)], out_hbm.at[c, pl.ds(s * rpt, rpt)])
    return k


_sc_scat_edges32 = _make_sc_scatter1(NP, 32, E, False, 200)
_sc_scat_ones_src = _make_sc_scatter1(NP, 16, E, True, 1000)
_sc_scat_node16 = _make_sc_scatter1(128, 16, NPS, False, 1600)
_sc_scat_ones_b = _make_sc_scatter1(128, 16, NPS, True, 1600)


def _dot(a, b):
    return jax.lax.dot(a, b, precision=_HIGH)


def _ek1_body(ghp, ggp, eap, uep, kh, kg, ke, ku, b1, w2, b2,
              w3g, w3e, b3, w4, b4, e2o, outo, sqo):
    z1 = (_dot(ghp[...], kh[...]) + _dot(ggp[...], kg[...]) +
          _dot(eap[...], ke[...]) + _dot(uep[...], ku[...]) + b1[...])
    e2 = _dot(_leaky(z1), w2[...]) + b2[...]
    z3 = _dot(ggp[...], w3g[...]) + _dot(e2, w3e[...]) + b3[...]
    out = _dot(_leaky(z3), w4[...]) + b4[...]
    e2o[...] = e2
    outo[...] = out
    sqo[...] = out * out


def _tc_ek1(ghp, ggp, eap, uep, kh, kg, ke, ku, b1, w2, b2, w3g, w3e, b3, w4, b4):
    n = TB // TBB
    row = lambda i: (i, 0)
    fix = lambda i: (0, 0)
    bs = lambda shp, im: pl.BlockSpec(shp, im)
    return pl.pallas_call(
        _ek1_body,
        grid=(n,),
        in_specs=[bs((TBB, 256), row)] * 4 + [
            bs((256, 256), fix), bs((256, 256), fix), bs((256, 256), fix),
            bs((256, 256), fix), bs((1, 256), fix), bs((256, 256), fix),
            bs((1, 256), fix), bs((256, 512), fix), bs((256, 512), fix),
            bs((1, 512), fix), bs((512, 512), fix), bs((1, 512), fix)],
        out_specs=[bs((TBB, 256), row), bs((TBB, 512), row), bs((TBB, 512), row)],
        out_shape=[jax.ShapeDtypeStruct((TB, 256), jnp.float32),
                   jax.ShapeDtypeStruct((TB, 512), jnp.float32),
                   jax.ShapeDtypeStruct((TB, 512), jnp.float32)],
    )(ghp, ggp, eap, uep, kh, kg, ke, ku, b1, w2, b2, w3g, w3e, b3, w4, b4)


def _ekc_body(outp, agp, t3o, t4o):
    t = outp[...] - agp[...]
    t2 = t * t
    t3o[...] = t2 * t
    t4o[...] = t2 * t2


def _tc_ekc(outp, agp):
    n = TB // TBB
    bs = pl.BlockSpec((TBB, 512), lambda i: (i, 0))
    return pl.pallas_call(
        _ekc_body, grid=(n,), in_specs=[bs, bs], out_specs=[bs, bs],
        out_shape=[jax.ShapeDtypeStruct((TB, 512), jnp.float32)] * 2,
    )(outp, agp)


def _ek2_body(gh2p, e2p, k5h, k5e, b5, w6, b6, outgo, stato, accs):
    i = pl.program_id(0)
    z = _dot(gh2p[...], k5h[...]) + _dot(e2p[...], k5e[...]) + b5[...]
    outgo[...] = _dot(_leaky(z), w6[...]) + b6[...]
    e2 = e2p[...]

    @pl.when(i == 0)
    def _():
        accs[...] = jnp.zeros_like(accs)
    accs[0:1, :] += jnp.sum(e2, axis=0, keepdims=True)
    accs[1:2, :] += jnp.sum(e2 * e2, axis=0, keepdims=True)

    @pl.when(i == pl.num_programs(0) - 1)
    def _():
        stato[...] = accs[...]


def _tc_ek2(gh2p, e2p, k5h, k5e, b5, w6, b6):
    n = TB // TBB
    row = lambda i: (i, 0)
    fix = lambda i: (0, 0)
    bs = lambda shp, im: pl.BlockSpec(shp, im)
    return pl.pallas_call(
        _ek2_body, grid=(n,),
        in_specs=[bs((TBB, 256), row), bs((TBB, 256), row),
                  bs((256, 512), fix), bs((256, 512), fix), bs((1, 512), fix),
                  bs((512, 512), fix), bs((1, 512), fix)],
        out_specs=[bs((TBB, 512), row), bs((8, 256), fix)],
        out_shape=[jax.ShapeDtypeStruct((TB, 512), jnp.float32),
                   jax.ShapeDtypeStruct((8, 256), jnp.float32)],
        scratch_shapes=[pltpu.VMEM((8, 256), jnp.float32)],
    )(gh2p, e2p, k5h, k5e, b5, w6, b6)


def _nodea_body(sv, c0, c1, ao):
    cnt = jnp.maximum((c0[...] + c1[...])[:, :1], 1.0)
    ao[...] = sv[...] / cnt


def _tc_nodea(sv, c0, c1):
    n = NP // TN
    row = lambda i: (i, 0)
    return pl.pallas_call(
        _nodea_body, grid=(n,),
        in_specs=[pl.BlockSpec((TN, 32), row), pl.BlockSpec((TN, 16), row),
                  pl.BlockSpec((TN, 16), row)],
        out_specs=pl.BlockSpec((TN, 32), row),
        out_shape=jax.ShapeDtypeStruct((NP, 32), jnp.float32),
    )(sv, c0, c1)


def _nodeb_body(xh, c0, c1, ss, t3s, t4s, a, uh,
                ah, aa, ab, ac, ad, au, wn, b1, w2, b2, xo, sto, accs):
    i = pl.program_id(0)
    cnt = (c0[...] + c1[...])[:, :1]
    cm = jnp.maximum(cnt, 1.0)
    av = a[...]
    bv = jnp.sqrt(1e-6 + jnp.maximum(ss[...] / cm - av * av, 0.0))
    b2v = bv * bv
    cv = (t3s[...] / cm) / (b2v * bv)
    dv = (t4s[...] / cm) / (b2v * b2v)
    z = (_dot(xh[...], ah[...]) + _dot(av, aa[...]) + _dot(bv, ab[...]) +
         _dot(cv, ac[...]) + _dot(dv, ad[...]) + _dot(uh[...], au[...]) +
         cnt * wn[...] + b1[...])
    xn = _dot(_leaky(z), w2[...]) + b2[...]
    xo[...] = xn
    rows = i * TN + jax.lax.broadcasted_iota(jnp.int32, (TN, 1), 0)
    m = (rows < 50000).astype(jnp.float32)
    xm = xn * m

    @pl.when(i == 0)
    def _():
        accs[...] = jnp.zeros_like(accs)
    accs[0:1, :] += jnp.sum(xm, axis=0, keepdims=True)
    accs[1:2, :] += jnp.sum(xm * xn, axis=0, keepdims=True)

    @pl.when(i == pl.num_programs(0) - 1)
    def _():
        sto[...] = accs[...]


def _tc_nodeb(xh, c0, c1, ss, t3s, t4s, a, uh, ah, aa, ab, ac, ad, au, wn, b1, w2, b2):
    n = NP // TN
    row = lambda i: (i, 0)
    fix = lambda i: (0, 0)
    bs = lambda shp, im: pl.BlockSpec(shp, im)
    return pl.pallas_call(
        _nodeb_body, grid=(n,),
        in_specs=[bs((TN, 16), row), bs((TN, 16), row), bs((TN, 16), row),
                  bs((TN, 32), row), bs((TN, 32), row), bs((TN, 32), row),
                  bs((TN, 32), row), bs((TN, 16), row),
                  bs((16, 16), fix), bs((32, 16), fix), bs((32, 16), fix),
                  bs((32, 16), fix), bs((32, 16), fix), bs((16, 16), fix),
                  bs((1, 16), fix), bs((1, 16), fix), bs((16, 16), fix),
                  bs((1, 16), fix)],
        out_specs=[bs((TN, 16), row), bs((8, 16), fix)],
        out_shape=[jax.ShapeDtypeStruct((NP, 16), jnp.float32),
                   jax.ShapeDtypeStruct((8, 16), jnp.float32)],
        scratch_shapes=[pltpu.VMEM((8, 16), jnp.float32)],
    )(xh, c0, c1, ss, t3s, t4s, a, uh, ah, aa, ab, ac, ad, au, wn, b1, w2, b2)


def _nodec_body(xg, ag0, ag1, ug, axg, aag, aug, b1, w2, b2, xo, sto, accs):
    i = pl.program_id(0)
    agv = ag0[...] + ag1[...]
    z = (_dot(xg[...], axg[...]) + _dot(agv, aag[...]) +
         _dot(ug[...], aug[...]) + b1[...])
    xn = _dot(_leaky(z), w2[...]) + b2[...]
    xo[...] = xn
    rows = i * TN + jax.lax.broadcasted_iota(jnp.int32, (TN, 1), 0)
    m = (rows < 50000).astype(jnp.float32)
    xm = xn * m

    @pl.when(i == 0)
    def _():
        accs[...] = jnp.zeros_like(accs)
    accs[0:1, :] += jnp.sum(xm, axis=0, keepdims=True)
    accs[1:2, :] += jnp.sum(xm * xn, axis=0, keepdims=True)

    @pl.when(i == pl.num_programs(0) - 1)
    def _():
        sto[...] = accs[...]


def _tc_nodec(xg, ag0, ag1, ug, axg, aag, aug, b1, w2, b2):
    n = NP // TN
    row = lambda i: (i, 0)
    fix = lambda i: (0, 0)
    bs = lambda shp, im: pl.BlockSpec(shp, im)
    return pl.pallas_call(
        _nodec_body, grid=(n,),
        in_specs=[bs((TN, 16), row), bs((TN, 32), row), bs((TN, 32), row),
                  bs((TN, 16), row),
                  bs((16, 16), fix), bs((32, 16), fix), bs((16, 16), fix),
                  bs((1, 16), fix), bs((16, 16), fix), bs((1, 16), fix)],
        out_specs=[bs((TN, 16), row), bs((8, 16), fix)],
        out_shape=[jax.ShapeDtypeStruct((NP, 16), jnp.float32),
                   jax.ShapeDtypeStruct((8, 16), jnp.float32)],
        scratch_shapes=[pltpu.VMEM((8, 16), jnp.float32)],
    )(xg, ag0, ag1, ug, axg, aag, aug, b1, w2, b2)


def _glob_body(u, hs0, hs1, cb0, cb1, gs0, gs1, cg0, cg1,
               au, ahm, agm, b1, w2, b2, uo):
    hm = (hs0[...] + hs1[...])[:16] / jnp.maximum((cb0[...] + cb1[...])[:16, :1], 1.0)
    gm = (gs0[...] + gs1[...])[:16] / jnp.maximum((cg0[...] + cg1[...])[:16, :1], 1.0)
    z = _dot(u[...], au[...]) + _dot(hm, ahm[...]) + _dot(gm, agm[...]) + b1[...]
    uo[...] = _dot(_leaky(z), w2[...]) + b2[...]


def _tc_glob(u, hs0, hs1, cb0, cb1, gs0, gs1, cg0, cg1, au, ahm, agm, b1, w2, b2):
    full = lambda shp: pl.BlockSpec(shp, lambda: (0, 0))
    return pl.pallas_call(
        _glob_body,
        in_specs=[full((16, 16))] + [full((128, 16))] * 8 +
                 [full((16, 16)), full((16, 16)), full((16, 16)),
                  full((1, 16)), full((16, 16)), full((1, 16))],
        out_specs=full((16, 16)),
        out_shape=jax.ShapeDtypeStruct((16, 16), jnp.float32),
    )(u, hs0, hs1, cb0, cb1, gs0, gs1, cg0, cg1, au, ahm, agm, b1, w2, b2)


def _ekf_body(ghp, ggp, eap, uep, nz, kh, kg, ke, ku, b1, w2, b2, to):
    z1 = (_dot(ghp[...], kh[...]) + _dot(ggp[...], kg[...]) +
          _dot(eap[...], ke[...]) + _dot(uep[...], ku[...]) + b1[...])
    e_out = _leaky(z1) * w2[...] + b2[...]
    t = TMAX * jax.nn.sigmoid(e_out) + nz[...]
    it = jnp.floor(t)
    to[...] = it + jax.nn.sigmoid(SHARP * (t - 0.5 - it))


def _tc_ekf(ghp, ggp, eap, uep, nz, kh, kg, ke, ku, b1, w2, b2):
    n = TB // TBB
    row = lambda i: (i, 0)
    fix = lambda i: (0, 0)
    bs = lambda shp, im: pl.BlockSpec(shp, im)
    return pl.pallas_call(
        _ekf_body, grid=(n,),
        in_specs=[bs((TBB, 256), row)] * 4 + [bs((TBB, 16), row)] +
                 [bs((256, 16), fix)] * 4 +
                 [bs((1, 16), fix), bs((1, 16), fix), bs((1, 16), fix)],
        out_specs=bs((TBB, 16), row),
        out_shape=jax.ShapeDtypeStruct((TB, 16), jnp.float32),
    )(ghp, ggp, eap, uep, nz, kh, kg, ke, ku, b1, w2, b2)


_I16 = None


def _kron16(w):
    din, dout = w.shape
    z = jnp.zeros((16, din, 16, dout), w.dtype)
    idx = jnp.arange(16)
    z = z.at[idx, :, idx, :].set(w)
    return z.reshape(16 * din, 16 * dout)


def _pk(x):
    return x.reshape(TB, -1)


def _unpk(x, d):
    return x.reshape(E, d)


def _stats_to_affine(s1, s2, n, gamma, beta):
    m = s1 / n
    v = s2 / n - m * m
    sc = gamma / jnp.sqrt(v + 1e-5)
    return sc, beta - m * sc


def kernel(x_h, x_g, edge_index, edge_attr, u, batch_e, batch_h, batch_g, params):
    f32 = jnp.float32
    src = edge_index[0].astype(jnp.int32)
    tgt = edge_index[1].astype(jnp.int32)
    be = batch_e.astype(jnp.int32)
    pad_n = NP - x_h.shape[0]
    xh = jnp.pad(x_h.astype(f32), ((0, pad_n), (0, 0)))
    xg = jnp.pad(x_g.astype(f32), ((0, pad_n), (0, 0)))
    up = jnp.pad(u.astype(f32), ((0, 16 - u.shape[0]), (0, 0)))
    ea = edge_attr.astype(f32)
    bh_pad = jnp.pad(batch_h.astype(jnp.int32), (0, NPS - batch_h.shape[0]),
                     constant_values=127)
    bg_pad = jnp.pad(batch_g.astype(jnp.int32), (0, NPS - batch_g.shape[0]),
                     constant_values=127)

    z32 = jnp.zeros((NP // NS, 32), f32)
    z16 = jnp.zeros((NP // NS, 16), f32)
    z16s = jnp.zeros((128 // NS, 16), f32)
    ones1k = jnp.ones((1000, 16), f32)
    ones16c = jnp.ones((1600, 16), f32)

    cnt = _sc_scat_ones_src(ones1k, src, z16)
    c0, c1 = cnt[0], cnt[1]
    cbh = _sc_scat_ones_b(ones16c, bh_pad, z16s)
    cbg = _sc_scat_ones_b(ones16c, bg_pad, z16s)

    one = jnp.ones((16,), f32)
    zero = jnp.zeros((16,), f32)
    sh, th = one, zero
    sg, tg = one, zero
    se, te = one, zero

    eap = _pk(ea)
    n_real = jnp.float32(50000.0)

    for p in params["blocks"]:
        w1 = p["edge"][0]["W"]; b1 = p["edge"][0]["b"]
        w2 = p["edge"][1]["W"]; b2 = p["edge"][1]["b"]
        w3 = p["h1"][0]["W"]; b3 = p["h1"][0]["b"]
        w4 = p["h1"][1]["W"]; b4 = p["h1"][1]["b"]

        gh = _sc_gather16(xh, src)
        gg = _sc_gather16(xg, tgt)
        ue = _sc_gatherue(up, be)
        ghp, ggp, uep = _pk(gh), _pk(gg), _pk(ue)

        w1h = w1[:, 0:16].T * sh[:, None]
        w1g = w1[:, 16:32].T * sg[:, None]
        w1e = w1[:, 32:48].T * se[:, None]
        w1u = w1[:, 48:64].T
        bias1 = b1 + th @ w1[:, 0:16].T + tg @ w1[:, 16:32].T + te @ w1[:, 32:48].T
        kh = _kron16(w1h); kg = _kron16(w1g); ke = _kron16(w1e); ku = _kron16(w1u)
        b1p = jnp.tile(bias1, 16)[None, :]
        w2p = _kron16(w2.T); b2p = jnp.tile(b2, 16)[None, :]
        w3g = w3[:, 0:16].T * sg[:, None]
        w3e = w3[:, 16:32].T
        bias3 = b3 + tg @ w3[:, 0:16].T
        k3g = _kron16(w3g); k3e = _kron16(w3e)
        b3p = jnp.tile(bias3, 16)[None, :]
        w4p = _kron16(w4.T); b4p = jnp.tile(b4, 16)[None, :]

        e2p, outp, sqp = _tc_ek1(ghp, ggp, eap, uep, kh, kg, ke, ku, b1p,
                                 w2p, b2p, k3g, k3e, b3p, w4p, b4p)

        out_e = _unpk(outp, 32)
        sq_e = _unpk(sqp, 32)
        s1pair = _sc_scatter_pair(out_e, sq_e, src, z32)
        a = _tc_nodea(s1pair[0], c0, c1)
        ag = _sc_gather32(a, src)
        t3p, t4p = _tc_ekc(outp, _pk(ag))
        s2pair = _sc_scatter_pair(_unpk(t3p, 32), _unpk(t4p, 32), src, z32)

        uh, ug = _sc_gatheru2(up, bh_pad, bg_pad)

        w5 = p["h2"][0]["W"]; b5 = p["h2"][0]["b"]
        w6 = p["h2"][1]["W"]; b6 = p["h2"][1]["b"]
        ah = w5[:, 0:16].T * sh[:, None]
        wn = w5[:, 16:17].T
        aa = w5[:, 17:49].T
        ab = w5[:, 49:81].T
        ac = w5[:, 81:113].T
        ad = w5[:, 113:145].T
        au = w5[:, 145:161].T
        bias5 = (b5 + th @ w5[:, 0:16].T)[None, :]
        xhn, sth = _tc_nodeb(xh, c0, c1, s1pair[1], s2pair[0], s2pair[1], a,
                             uh[:NP], ah, aa, ab, ac, ad, au, wn, bias5,
                             w6.T, b6[None, :])

        gh2 = _sc_gather16(xhn, src)
        w7 = p["g1"][0]["W"]; b7 = p["g1"][0]["b"]
        w8 = p["g1"][1]["W"]; b8 = p["g1"][1]["b"]
        k7h = _kron16(w7[:, 0:16].T)
        k7e = _kron16(w7[:, 16:32].T)
        b7p = jnp.tile(b7, 16)[None, :]
        w8p = _kron16(w8.T); b8p = jnp.tile(b8, 16)[None, :]
        outgp, ste = _tc_ek2(_pk(gh2), e2p, k7h, k7e, b7p, w8p, b8p)
        agp_pair = _sc_scat_edges32(_unpk(outgp, 32), tgt, z32)

        w9 = p["g2"][0]["W"]; b9 = p["g2"][0]["b"]
        wa = p["g2"][1]["W"]; ba = p["g2"][1]["b"]
        axg = w9[:, 0:16].T * sg[:, None]
        aag = w9[:, 16:48].T
        aug = w9[:, 48:64].T
        bias9 = (b9 + tg @ w9[:, 0:16].T)[None, :]
        xgn, stg = _tc_nodec(xg, agp_pair[0], agp_pair[1], ug[:NP],
                             axg, aag, aug, bias9, wa.T, ba[None, :])

        xhn_s = jnp.pad(xhn, ((0, NPS - NP), (0, 0)))
        xgn_s = jnp.pad(xgn, ((0, NPS - NP), (0, 0)))
        hsum = _sc_scat_node16(xhn_s, bh_pad, z16s)
        gsum = _sc_scat_node16(xgn_s, bg_pad, z16s)
        wg1 = p["glob"][0]["W"]; bg1 = p["glob"][0]["b"]
        wg2 = p["glob"][1]["W"]; bg2 = p["glob"][1]["b"]
        au_g = wg1[:, 0:16].T
        ahm = wg1[:, 16:32].T
        agm = wg1[:, 32:48].T
        u_new = _tc_glob(up, hsum[0], hsum[1], cbh[0], cbh[1],
                         gsum[0], gsum[1], cbg[0], cbg[1],
                         au_g, ahm, agm, bg1[None, :], wg2.T, bg2[None, :])

        es = ste[0].reshape(16, 16).sum(axis=0)
        ess = ste[1].reshape(16, 16).sum(axis=0)
        sh, th = _stats_to_affine(sth[0], sth[1], n_real, p["bn_xh"]["gamma"], p["bn_xh"]["beta"])
        sg, tg = _stats_to_affine(stg[0], stg[1], n_real, p["bn_xg"]["gamma"], p["bn_xg"]["beta"])
        se, te = _stats_to_affine(es, ess, jnp.float32(E), p["bn_e"]["gamma"], p["bn_e"]["beta"])

        xh, xg, up = xhn, xgn, u_new
        eap = e2p

    gh = _sc_gather16(xh, src)
    gg = _sc_gather16(xg, tgt)
    ue = _sc_gatherue(up, be)
    wl1 = params["last"][0]["W"]; bl1 = params["last"][0]["b"]
    wl2 = params["last"][1]["W"]; bl2 = params["last"][1]["b"]
    khf = _kron16(wl1[:, 0:16].T * sh[:, None])
    kgf = _kron16(wl1[:, 16:32].T * sg[:, None])
    kef = _kron16(wl1[:, 32:48].T * se[:, None])
    kuf = _kron16(wl1[:, 48:64].T)
    biasf = (bl1 + th @ wl1[:, 0:16].T + tg @ wl1[:, 16:32].T
             + te @ wl1[:, 32:48].T)
    b1f = jnp.tile(biasf, 16)[None, :]
    w2f = jnp.tile(wl2[0], 16)[None, :]
    b2f = jnp.tile(bl2, 16)[None, :]
    noise = NOISE * (jax.random.uniform(jax.random.key(42), (E,), dtype=f32) - 0.5)
    tpk = _tc_ekf(_pk(gh), _pk(gg), eap, _pk(ue), noise.reshape(TB, 16),
                  khf, kgf, kef, kuf, b1f, w2f, b2f)
    return (tpk.reshape(E), edge_index)
